# baseline (speedup 1.0000x reference)
import functools

import numpy as np

import jax
import jax.numpy as jnp
from jax import lax
from jax.experimental import pallas as pl
from jax.experimental.pallas import tpu as pltpu
from jax.experimental.pallas import tpu_sc as plsc

B, T, DIM = 2, 50000, 3
HID, CD = 128, 32
RESO = 32
S = RESO ** 3
NB = 5
BT = B * T
ROWS = 2000
GRID = BT // ROWS


def _relu(x):
    return jnp.maximum(x, 0.0)


def _dot(a, b):
    return jnp.dot(a, b, preferred_element_type=jnp.float32)


def _head_body(pts_ref, wp_ref, bp_ref, w0_ref, b0_ref, w1_ref, b1_ref,
               ws_ref, out_ref):
    p = pts_ref[...]
    h = _dot(p, wp_ref[...]) + bp_ref[...]
    net = _dot(_relu(h), w0_ref[...]) + b0_ref[...]
    dx = _dot(_relu(net), w1_ref[...]) + b1_ref[...]
    out_ref[...] = _dot(h, ws_ref[...]) + dx


def _block_body(net_ref, pooled_ref, w0_ref, b0_ref, w1_ref, b1_ref,
                ws_ref, out_ref):
    net = net_ref[...]
    pooled = pooled_ref[...]
    w0 = w0_ref[...]
    ws = ws_ref[...]
    h = (_dot(_relu(net), w0[:HID]) + _dot(_relu(pooled), w0[HID:])
         + b0_ref[...])
    dx = _dot(_relu(h), w1_ref[...]) + b1_ref[...]
    out_ref[...] = _dot(net, ws[:HID]) + _dot(pooled, ws[HID:]) + dx


def _block_last_body(net_ref, pooled_ref, w0_ref, b0_ref, w1_ref, b1_ref,
                     ws_ref, wc_ref, bc_ref, out_ref):
    net = net_ref[...]
    pooled = pooled_ref[...]
    w0 = w0_ref[...]
    ws = ws_ref[...]
    h = (_dot(_relu(net), w0[:HID]) + _dot(_relu(pooled), w0[HID:])
         + b0_ref[...])
    dx = _dot(_relu(h), w1_ref[...]) + b1_ref[...]
    out = _dot(net, ws[:HID]) + _dot(pooled, ws[HID:]) + dx
    out_ref[...] = _dot(out, wc_ref[...]) + bc_ref[...]


def _row_spec(cols):
    return pl.BlockSpec((ROWS, cols), lambda i: (i, 0))


def _full_spec(shape):
    nd = len(shape)
    return pl.BlockSpec(shape, lambda i: (0,) * nd)


def _run_head(pts, wp, bp, w0, b0, w1, b1, ws):
    return pl.pallas_call(
        _head_body,
        grid=(GRID,),
        in_specs=[_row_spec(DIM), _full_spec(wp.shape), _full_spec(bp.shape),
                  _full_spec(w0.shape), _full_spec(b0.shape),
                  _full_spec(w1.shape), _full_spec(b1.shape),
                  _full_spec(ws.shape)],
        out_specs=_row_spec(HID),
        out_shape=jax.ShapeDtypeStruct((BT, HID), jnp.float32),
    )(pts, wp, bp, w0, b0, w1, b1, ws)


def _run_block(net, pooled, w0, b0, w1, b1, ws):
    return pl.pallas_call(
        _block_body,
        grid=(GRID,),
        in_specs=[_row_spec(HID), _row_spec(HID),
                  _full_spec(w0.shape), _full_spec(b0.shape),
                  _full_spec(w1.shape), _full_spec(b1.shape),
                  _full_spec(ws.shape)],
        out_specs=_row_spec(HID),
        out_shape=jax.ShapeDtypeStruct((BT, HID), jnp.float32),
    )(net, pooled, w0, b0, w1, b1, ws)


def _run_block_last(net, pooled, w0, b0, w1, b1, ws, wc, bc):
    return pl.pallas_call(
        _block_last_body,
        grid=(GRID,),
        in_specs=[_row_spec(HID), _row_spec(HID),
                  _full_spec(w0.shape), _full_spec(b0.shape),
                  _full_spec(w1.shape), _full_spec(b1.shape),
                  _full_spec(ws.shape), _full_spec(wc.shape),
                  _full_spec(bc.shape)],
        out_specs=_row_spec(HID),
        out_shape=jax.ShapeDtypeStruct((BT, HID), jnp.float32),
    )(net, pooled, w0, b0, w1, b1, ws, wc, bc)


NC, NS, L = 2, 16, 16
NW = NC * NS
SEGBLK = 512
MBLK = 1024
CCHUNK = 512
NEG_INF = float('-inf')


_DN = lax.GatherDimensionNumbers(offset_dims=(), collapsed_slice_dims=(0,),
                                 start_index_map=(0,))


def _vgather(vec, idxv):
    return lax.gather(vec, idxv.reshape(L, 1), _DN, (1,),
                      mode=lax.GatherScatterMode.PROMISE_IN_BOUNDS)


def _sc_segmax_body(vals_hbm, scode_hbm, bounds_hbm, iota_hbm, seg_hbm,
                    bnd_buf, iv_buf, sc_chunk, rows_buf, acc, sem):
    wid = lax.axis_index("s") * NC + lax.axis_index("c")
    woff = pl.multiple_of(wid * L, 8)
    pltpu.sync_copy(bounds_hbm.at[pl.ds(woff, L)], bnd_buf)
    pltpu.sync_copy(iota_hbm, iv_buf)
    bv = bnd_buf[...]
    iv = iv_buf[...]
    zi = iv ^ iv
    neg_row = zi.astype(jnp.float32) + NEG_INF

    for t in range(2 * B):
        b = t // 2
        lo = ((t % 2) * NW + wid) * SEGBLK
        j0 = bv[2 * t]
        j1 = bv[2 * t + 1]

        def initrow(i, _):
            acc[pl.ds(i * L, L)] = neg_row
            return 0
        lax.fori_loop(0, (SEGBLK + 1) * HID // L, initrow, 0)

        ja = (j0 // 8) * 8
        nchk = (j1 - ja + CCHUNK - 1) // CCHUNK

        def chunkf(ci, _):
            base_c = ja + ci * CCHUNK
            coff = pl.multiple_of(base_c, 8)
            pltpu.sync_copy(scode_hbm.at[pl.ds(coff, CCHUNK)], sc_chunk)
            ng = jnp.minimum(CCHUNK // L, (j1 - base_c + L - 1) // L)

            def grp(gi, _):
                code16 = sc_chunk[pl.ds(gi * L, L)]
                pid16 = (code16 & 0xFFFF) + b * T
                s16 = (code16 >> 16) - (zi + lo)
                pltpu.async_copy(vals_hbm.at[pid16], rows_buf, sem).wait()
                base = base_c + gi * L
                j0v = zi + j0
                j1v = zi + j1
                dumprow = zi + SEGBLK
                for jj in range(L):
                    posv = zi + (base + jj)
                    mjv = (posv >= j0v) & (posv < j1v)
                    sb = jnp.where(mjv, _vgather(s16, zi + jj), dumprow)
                    sbase = sb * HID
                    for g in range(HID // L):
                        fl = sbase + (g * L + iv)
                        cur = plsc.load_gather(acc, [fl])
                        val = rows_buf[jj, pl.ds(g * L, L)]
                        plsc.store_scatter(acc, [fl],
                                           jnp.maximum(cur, val))
                return 0
            lax.fori_loop(0, ng, grp, 0)
            return 0
        lax.fori_loop(0, nchk, chunkf, 0)
        pltpu.sync_copy(acc.at[pl.ds(0, SEGBLK * HID)],
                        seg_hbm.at[pl.ds(b * S * HID + lo * HID,
                                         SEGBLK * HID)])


_GB_SUB = 128
_GB_PER_B = (T + _GB_SUB - 1) // _GB_SUB
_GB_TOTAL = B * _GB_PER_B
_GB_ITERS = (_GB_TOTAL + NW - 1) // NW


def _sc_gather_body(seg_hbm, idx_hbm, out_hbm, idx_row, rows_buf, sem):
    wid = lax.axis_index("s") * NC + lax.axis_index("c")

    def it(i, _):
        tid = wid + i * NW

        @pl.when(tid < _GB_TOTAL)
        def _():
            bb = tid // _GB_PER_B
            j = tid % _GB_PER_B
            start = jnp.where(j == _GB_PER_B - 1, T - _GB_SUB, j * _GB_SUB)
            off = pl.multiple_of(bb * T + start, 8)
            pltpu.sync_copy(idx_hbm.at[pl.ds(off, _GB_SUB)],
                            idx_row.at[0])
            pltpu.async_copy(seg_hbm.at[bb].at[idx_row.at[0]],
                             rows_buf, sem).wait()
            pltpu.sync_copy(rows_buf,
                            out_hbm.at[pl.ds(bb * T + start, _GB_SUB), :])
        return 0
    lax.fori_loop(0, _GB_ITERS, it, 0)


def _sc_mean_body(c_hbm, scode_hbm, bounds_hbm, iota_hbm, fea_hbm, bnd_buf,
                  iv_buf, sc_chunk, crows, acc, cnt_buf, sem):
    wid = lax.axis_index("s") * NC + lax.axis_index("c")
    woff = pl.multiple_of(wid * L, 8)
    pltpu.sync_copy(bounds_hbm.at[pl.ds(woff, L)], bnd_buf)
    pltpu.sync_copy(iota_hbm, iv_buf)
    bv = bnd_buf[...]
    iv = iv_buf[...]
    zi = iv ^ iv
    zrow = zi.astype(jnp.float32)
    lane0 = iv < 1

    for b in range(B):
        lo = wid * MBLK
        j0 = bv[2 * b]
        j1 = bv[2 * b + 1]

        def initrow(i, _):
            acc[pl.ds(i * L, L)] = zrow
            return 0
        lax.fori_loop(0, CD * (MBLK + L) // L, initrow, 0)

        def initcnt(i, _):
            cnt_buf[pl.ds(i * L, L)] = zrow
            return 0
        lax.fori_loop(0, (MBLK + L) // L, initcnt, 0)

        ja = (j0 // 8) * 8
        nchk = (j1 - ja + CCHUNK - 1) // CCHUNK

        def chunkf(ci, _):
            base_c = ja + ci * CCHUNK
            coff = pl.multiple_of(base_c, 8)
            pltpu.sync_copy(scode_hbm.at[pl.ds(coff, CCHUNK)], sc_chunk)
            ng = jnp.minimum(CCHUNK // L, (j1 - base_c + L - 1) // L)

            def grp(gi, _):
                code16 = sc_chunk[pl.ds(gi * L, L)]
                pid16 = (code16 & 0xFFFF) + b * T
                s16 = (code16 >> 16) - (zi + lo)
                pltpu.async_copy(c_hbm.at[pid16], crows, sem).wait()
                base = base_c + gi * L
                j0v = zi + j0
                j1v = zi + j1
                dumpcol = zi + MBLK
                for jj in range(L):
                    posv = zi + (base + jj)
                    mjv = (posv >= j0v) & (posv < j1v)
                    sb = jnp.where(mjv, _vgather(s16, zi + jj), dumpcol)
                    for g in range(CD // L):
                        fl = (g * L + iv) * (MBLK + L) + sb
                        cur = plsc.load_gather(acc, [fl])
                        val = crows[jj, pl.ds(g * L, L)]
                        plsc.store_scatter(acc, [fl], cur + val)
                    sb0 = jnp.where(lane0, sb, dumpcol)
                    ccur = plsc.load_gather(cnt_buf, [sb0])
                    plsc.store_scatter(cnt_buf, [sb0], ccur + 1.0)
                return 0
            lax.fori_loop(0, ng, grp, 0)
            return 0
        lax.fori_loop(0, nchk, chunkf, 0)

        def fin(sg, _):
            cv = cnt_buf[pl.ds(sg * L, L)]
            inv = 1.0 / jnp.maximum(cv, 1.0)
            for ch in range(CD):
                o = ch * (MBLK + L)
                acc[pl.ds(o + sg * L, L)] = acc[pl.ds(o + sg * L, L)] * inv
            return 0
        lax.fori_loop(0, MBLK // L, fin, 0)
        for ch in range(CD):
            pltpu.sync_copy(acc.at[pl.ds(ch * (MBLK + L), MBLK)],
                            fea_hbm.at[pl.ds(b * CD * S + ch * S + lo, MBLK)])


@functools.lru_cache(maxsize=None)
def _sc_fns():
    mesh = plsc.VectorSubcoreMesh(core_axis_name="c", subcore_axis_name="s",
                                  num_cores=NC, num_subcores=NS)
    cparams = pltpu.CompilerParams(needs_layout_passes=False)
    segmax = pl.kernel(
        _sc_segmax_body, mesh=mesh, compiler_params=cparams,
        out_type=jax.ShapeDtypeStruct((B * S * HID,), jnp.float32),
        scratch_types=[
            pltpu.VMEM((L,), jnp.int32),
            pltpu.VMEM((L,), jnp.int32),
            pltpu.VMEM((CCHUNK,), jnp.int32),
            pltpu.VMEM((L, HID), jnp.float32),
            pltpu.VMEM(((SEGBLK + 1) * HID,), jnp.float32),
            pltpu.SemaphoreType.DMA,
        ])
    gather = pl.kernel(
        _sc_gather_body, mesh=mesh, compiler_params=cparams,
        out_type=jax.ShapeDtypeStruct((BT, HID), jnp.float32),
        scratch_types=[
            pltpu.VMEM((1, _GB_SUB), jnp.int32),
            pltpu.VMEM((_GB_SUB, HID), jnp.float32),
            pltpu.SemaphoreType.DMA,
        ])
    mean = pl.kernel(
        _sc_mean_body, mesh=mesh, compiler_params=cparams,
        out_type=jax.ShapeDtypeStruct((B * CD * S,), jnp.float32),
        scratch_types=[
            pltpu.VMEM((L,), jnp.int32),
            pltpu.VMEM((L,), jnp.int32),
            pltpu.VMEM((CCHUNK,), jnp.int32),
            pltpu.VMEM((L, HID), jnp.float32),
            pltpu.VMEM((CD * (MBLK + L),), jnp.float32),
            pltpu.VMEM((MBLK + L,), jnp.float32),
            pltpu.SemaphoreType.DMA,
        ])
    return segmax, gather, mean


def _route(index):
    idxf = index[:, 0, :].astype(jnp.int32)
    pid = lax.broadcasted_iota(jnp.int32, (B, T), 1)
    scode_bt = jnp.sort((idxf << 16) | pid, axis=1)
    scode = jnp.concatenate(
        [scode_bt.reshape(BT), jnp.zeros((CCHUNK,), jnp.int32)])

    w = np.arange(NW, dtype=np.int64)
    los = (np.arange(2, dtype=np.int64)[:, None] * NW + w[None, :]) * SEGBLK
    edges = np.minimum((np.stack([los, los + SEGBLK], axis=-1) << 16),
                       np.int64(2**31 - 1)).reshape(-1).astype(np.int32)
    edges = jnp.asarray(edges)
    jm = jax.vmap(lambda sc: jnp.searchsorted(sc, edges))(scode_bt)
    jm = (jm.astype(jnp.int32).reshape(B, 2, NW, 2)
          + (jnp.arange(B, dtype=jnp.int32) * T)[:, None, None, None])
    bm = jnp.transpose(jm, (2, 0, 1, 3)).reshape(NW, 8)
    bounds_max = jnp.concatenate(
        [bm, jnp.zeros((NW, 8), jnp.int32)], axis=1).reshape(NW * L)

    mlos = w * MBLK
    medges = np.minimum((np.stack([mlos, mlos + MBLK], axis=-1) << 16),
                        np.int64(2**31 - 1)).reshape(-1).astype(np.int32)
    medges = jnp.asarray(medges)
    jq = jax.vmap(lambda sc: jnp.searchsorted(sc, medges))(scode_bt)
    jq = (jq.astype(jnp.int32).reshape(B, NW, 2)
          + (jnp.arange(B, dtype=jnp.int32) * T)[:, None, None])
    bq = jnp.transpose(jq, (1, 0, 2)).reshape(NW, 4)
    bounds_mean = jnp.concatenate(
        [bq, jnp.zeros((NW, 12), jnp.int32)], axis=1).reshape(NW * L)

    iota = jnp.arange(L, dtype=jnp.int32)
    return idxf.reshape(BT), scode, bounds_max, bounds_mean, iota


def _conv3d(x, w, b):
    y = jax.lax.conv_general_dilated(
        x, w, (1, 1, 1), 'SAME', dimension_numbers=('NCDHW', 'OIDHW', 'NCDHW'))
    return y + b[None, :, None, None, None]


def _maxpool(x):
    return jax.lax.reduce_window(x, -jnp.inf, jax.lax.max, (1, 1, 2, 2, 2),
                                 (1, 1, 2, 2, 2), 'VALID')


def _upsample(x):
    x = jnp.repeat(x, 2, axis=2)
    x = jnp.repeat(x, 2, axis=3)
    x = jnp.repeat(x, 2, axis=4)
    return x


def kernel(points, index, fc_pos_w, fc_pos_b, blk_fc0_w, blk_fc0_b,
           blk_fc1_w, blk_fc1_b, blk_sc_w, fc_c_w, fc_c_b,
           u_e1a_w, u_e1a_b, u_e1b_w, u_e1b_b, u_e2a_w, u_e2a_b,
           u_e2b_w, u_e2b_b, u_d1a_w, u_d1a_b, u_d1b_w, u_d1b_b,
           u_out_w, u_out_b):
    idx, scode, bounds_max, bounds_mean, iota = _route(index)
    _sc_segmax, _sc_gather, _sc_mean = _sc_fns()
    pts = points.reshape(BT, DIM)
    bp = fc_pos_b.reshape(1, 2 * HID)

    net = _run_head(pts, fc_pos_w, bp, blk_fc0_w[0],
                    blk_fc0_b[0].reshape(1, HID), blk_fc1_w[0],
                    blk_fc1_b[0].reshape(1, HID), blk_sc_w[0])

    for i in range(1, NB - 1):
        seg = _sc_segmax(net, scode, bounds_max, iota).reshape(B, S, HID)
        pooled = _sc_gather(seg, idx)
        net = _run_block(net, pooled, blk_fc0_w[i],
                         blk_fc0_b[i].reshape(1, HID), blk_fc1_w[i],
                         blk_fc1_b[i].reshape(1, HID), blk_sc_w[i])

    seg = _sc_segmax(net, scode, bounds_max, iota).reshape(B, S, HID)
    pooled = _sc_gather(seg, idx)
    wc_p = jnp.pad(fc_c_w, ((0, 0), (0, HID - CD)))
    bc_p = jnp.pad(fc_c_b, (0, HID - CD)).reshape(1, HID)
    c = _run_block_last(net, pooled, blk_fc0_w[NB - 1],
                        blk_fc0_b[NB - 1].reshape(1, HID), blk_fc1_w[NB - 1],
                        blk_fc1_b[NB - 1].reshape(1, HID), blk_sc_w[NB - 1],
                        wc_p, bc_p)

    fea = _sc_mean(c, scode, bounds_mean, iota).reshape(
        B, CD, RESO, RESO, RESO)
    e1 = jax.nn.relu(_conv3d(jax.nn.relu(_conv3d(fea, u_e1a_w, u_e1a_b)),
                             u_e1b_w, u_e1b_b))
    p = _maxpool(e1)
    e2 = jax.nn.relu(_conv3d(jax.nn.relu(_conv3d(p, u_e2a_w, u_e2a_b)),
                             u_e2b_w, u_e2b_b))
    u = _upsample(e2)
    d = jnp.concatenate([u, e1], axis=1)
    d = jax.nn.relu(_conv3d(jax.nn.relu(_conv3d(d, u_d1a_w, u_d1a_b)),
                            u_d1b_w, u_d1b_b))
    out = _conv3d(d, u_out_w, u_out_b)
    return out


# speedup vs baseline: 1.1535x; 1.1535x over previous
import functools

import numpy as np

import jax
import jax.numpy as jnp
from jax import lax
from jax.experimental import pallas as pl
from jax.experimental.pallas import tpu as pltpu
from jax.experimental.pallas import tpu_sc as plsc

B, T, DIM = 2, 50000, 3
HID, CD = 128, 32
RESO = 32
S = RESO ** 3
NB = 5
BT = B * T
ROWS = 2000
GRID = BT // ROWS


def _relu(x):
    return jnp.maximum(x, 0.0)


def _dot(a, b):
    return jnp.dot(a, b, preferred_element_type=jnp.float32)


def _head_body(pts_ref, wp_ref, bp_ref, w0_ref, b0_ref, w1_ref, b1_ref,
               ws_ref, out_ref):
    p = pts_ref[...]
    h = _dot(p, wp_ref[...]) + bp_ref[...]
    net = _dot(_relu(h), w0_ref[...]) + b0_ref[...]
    dx = _dot(_relu(net), w1_ref[...]) + b1_ref[...]
    out_ref[...] = _dot(h, ws_ref[...]) + dx


def _block_body(net_ref, pooled_ref, w0_ref, b0_ref, w1_ref, b1_ref,
                ws_ref, out_ref):
    net = net_ref[...]
    pooled = pooled_ref[...]
    w0 = w0_ref[...]
    ws = ws_ref[...]
    h = (_dot(_relu(net), w0[:HID]) + _dot(_relu(pooled), w0[HID:])
         + b0_ref[...])
    dx = _dot(_relu(h), w1_ref[...]) + b1_ref[...]
    out_ref[...] = _dot(net, ws[:HID]) + _dot(pooled, ws[HID:]) + dx


def _block_last_body(net_ref, pooled_ref, w0_ref, b0_ref, w1_ref, b1_ref,
                     ws_ref, wc_ref, bc_ref, out_ref):
    net = net_ref[...]
    pooled = pooled_ref[...]
    w0 = w0_ref[...]
    ws = ws_ref[...]
    h = (_dot(_relu(net), w0[:HID]) + _dot(_relu(pooled), w0[HID:])
         + b0_ref[...])
    dx = _dot(_relu(h), w1_ref[...]) + b1_ref[...]
    out = _dot(net, ws[:HID]) + _dot(pooled, ws[HID:]) + dx
    out_ref[...] = _dot(out, wc_ref[...]) + bc_ref[...]


def _row_spec(cols):
    return pl.BlockSpec((ROWS, cols), lambda i: (i, 0))


def _full_spec(shape):
    nd = len(shape)
    return pl.BlockSpec(shape, lambda i: (0,) * nd)


def _run_head(pts, wp, bp, w0, b0, w1, b1, ws):
    return pl.pallas_call(
        _head_body,
        grid=(GRID,),
        in_specs=[_row_spec(DIM), _full_spec(wp.shape), _full_spec(bp.shape),
                  _full_spec(w0.shape), _full_spec(b0.shape),
                  _full_spec(w1.shape), _full_spec(b1.shape),
                  _full_spec(ws.shape)],
        out_specs=_row_spec(HID),
        out_shape=jax.ShapeDtypeStruct((BT, HID), jnp.float32),
    )(pts, wp, bp, w0, b0, w1, b1, ws)


def _run_block(net, pooled, w0, b0, w1, b1, ws):
    return pl.pallas_call(
        _block_body,
        grid=(GRID,),
        in_specs=[_row_spec(HID), _row_spec(HID),
                  _full_spec(w0.shape), _full_spec(b0.shape),
                  _full_spec(w1.shape), _full_spec(b1.shape),
                  _full_spec(ws.shape)],
        out_specs=_row_spec(HID),
        out_shape=jax.ShapeDtypeStruct((BT, HID), jnp.float32),
    )(net, pooled, w0, b0, w1, b1, ws)


def _run_block_last(net, pooled, w0, b0, w1, b1, ws, wc, bc):
    return pl.pallas_call(
        _block_last_body,
        grid=(GRID,),
        in_specs=[_row_spec(HID), _row_spec(HID),
                  _full_spec(w0.shape), _full_spec(b0.shape),
                  _full_spec(w1.shape), _full_spec(b1.shape),
                  _full_spec(ws.shape), _full_spec(wc.shape),
                  _full_spec(bc.shape)],
        out_specs=_row_spec(HID),
        out_shape=jax.ShapeDtypeStruct((BT, HID), jnp.float32),
    )(net, pooled, w0, b0, w1, b1, ws, wc, bc)


NC, NS, L = 2, 16, 16
NW = NC * NS
SEGBLK = 512
MBLK = 1024
CCHUNK = 512
MG = 128
NEG_INF = float('-inf')


_DN = lax.GatherDimensionNumbers(offset_dims=(), collapsed_slice_dims=(0,),
                                 start_index_map=(0,))


def _vgather(vec, idxv):
    return lax.gather(vec, idxv.reshape(L, 1), _DN, (1,),
                      mode=lax.GatherScatterMode.PROMISE_IN_BOUNDS)


def _sc_segmax_body(vals_hbm, scode_hbm, bounds_hbm, iota_hbm, seg_hbm,
                    bnd_buf, iv_buf, sc_chunk, pid_buf, rows_buf, acc, sem):
    wid = lax.axis_index("s") * NC + lax.axis_index("c")
    woff = pl.multiple_of(wid * L, 8)
    pltpu.sync_copy(bounds_hbm.at[pl.ds(woff, L)], bnd_buf)
    pltpu.sync_copy(iota_hbm, iv_buf)
    bv = bnd_buf[...]
    iv = iv_buf[...]
    zi = iv ^ iv
    neg_row = zi.astype(jnp.float32) + NEG_INF

    for t in range(2 * B):
        b = t // 2
        lo = ((t % 2) * NW + wid) * SEGBLK
        j0 = bv[2 * t]
        j1 = bv[2 * t + 1]

        def initrow(i, _):
            acc[pl.ds(i * L, L)] = neg_row
            return 0
        lax.fori_loop(0, (SEGBLK + 1) * HID // L, initrow, 0)

        ja = (j0 // 8) * 8
        nchk = (j1 - ja + CCHUNK - 1) // CCHUNK

        def chunkf(ci, _):
            base_c = ja + ci * CCHUNK
            coff = pl.multiple_of(base_c, 8)
            pltpu.sync_copy(scode_hbm.at[pl.ds(coff, CCHUNK)], sc_chunk)
            nmg = jnp.minimum(CCHUNK // MG, (j1 - base_c + MG - 1) // MG)

            def macro(mg, _):
                for q in range(MG // L):
                    cod = sc_chunk[pl.ds(mg * MG + q * L, L)]
                    pid_buf[0, pl.ds(q * L, L)] = (cod & 0xFFFF) + b * T
                pltpu.async_copy(vals_hbm.at[pid_buf.at[0]], rows_buf,
                                 sem).wait()
                ng = jnp.minimum(MG // L,
                                 (j1 - (base_c + mg * MG) + L - 1) // L)

                def grp(gi, _):
                    code16 = sc_chunk[pl.ds(mg * MG + gi * L, L)]
                    s16 = (code16 >> 16) - (zi + lo)
                    base = base_c + mg * MG + gi * L
                    j0v = zi + j0
                    j1v = zi + j1
                    dumprow = zi + SEGBLK
                    for jj in range(L):
                        posv = zi + (base + jj)
                        mjv = (posv >= j0v) & (posv < j1v)
                        sb = jnp.where(mjv, _vgather(s16, zi + jj), dumprow)
                        sbase = sb * HID
                        rr = gi * L + jj
                        for g in range(HID // L):
                            fl = sbase + (g * L + iv)
                            cur = plsc.load_gather(acc, [fl])
                            val = rows_buf[rr, pl.ds(g * L, L)]
                            plsc.store_scatter(acc, [fl],
                                               jnp.maximum(cur, val))
                    return 0
                lax.fori_loop(0, ng, grp, 0)
                return 0
            lax.fori_loop(0, nmg, macro, 0)
            return 0
        lax.fori_loop(0, nchk, chunkf, 0)
        pltpu.sync_copy(acc.at[pl.ds(0, SEGBLK * HID)],
                        seg_hbm.at[pl.ds(b * S * HID + lo * HID,
                                         SEGBLK * HID)])


_GB_SUB = 128
_GB_PER_B = (T + _GB_SUB - 1) // _GB_SUB
_GB_TOTAL = B * _GB_PER_B
_GB_ITERS = (_GB_TOTAL + NW - 1) // NW


def _sc_gather_body(seg_hbm, idx_hbm, out_hbm, idx_row, rows_buf, sem):
    wid = lax.axis_index("s") * NC + lax.axis_index("c")

    def it(i, _):
        tid = wid + i * NW

        @pl.when(tid < _GB_TOTAL)
        def _():
            bb = tid // _GB_PER_B
            j = tid % _GB_PER_B
            start = jnp.where(j == _GB_PER_B - 1, T - _GB_SUB, j * _GB_SUB)
            off = pl.multiple_of(bb * T + start, 8)
            pltpu.sync_copy(idx_hbm.at[pl.ds(off, _GB_SUB)],
                            idx_row.at[0])
            pltpu.async_copy(seg_hbm.at[bb].at[idx_row.at[0]],
                             rows_buf, sem).wait()
            pltpu.sync_copy(rows_buf,
                            out_hbm.at[pl.ds(bb * T + start, _GB_SUB), :])
        return 0
    lax.fori_loop(0, _GB_ITERS, it, 0)


def _sc_mean_body(c_hbm, scode_hbm, bounds_hbm, iota_hbm, fea_hbm, bnd_buf,
                  iv_buf, sc_chunk, pid_buf, crows, acc, cnt_buf, sem):
    wid = lax.axis_index("s") * NC + lax.axis_index("c")
    woff = pl.multiple_of(wid * L, 8)
    pltpu.sync_copy(bounds_hbm.at[pl.ds(woff, L)], bnd_buf)
    pltpu.sync_copy(iota_hbm, iv_buf)
    bv = bnd_buf[...]
    iv = iv_buf[...]
    zi = iv ^ iv
    zrow = zi.astype(jnp.float32)
    lane0 = iv < 1

    for b in range(B):
        lo = wid * MBLK
        j0 = bv[2 * b]
        j1 = bv[2 * b + 1]

        def initrow(i, _):
            acc[pl.ds(i * L, L)] = zrow
            return 0
        lax.fori_loop(0, CD * (MBLK + L) // L, initrow, 0)

        def initcnt(i, _):
            cnt_buf[pl.ds(i * L, L)] = zrow
            return 0
        lax.fori_loop(0, (MBLK + L) // L, initcnt, 0)

        ja = (j0 // 8) * 8
        nchk = (j1 - ja + CCHUNK - 1) // CCHUNK

        def chunkf(ci, _):
            base_c = ja + ci * CCHUNK
            coff = pl.multiple_of(base_c, 8)
            pltpu.sync_copy(scode_hbm.at[pl.ds(coff, CCHUNK)], sc_chunk)
            nmg = jnp.minimum(CCHUNK // MG, (j1 - base_c + MG - 1) // MG)

            def macro(mg, _):
                for q in range(MG // L):
                    cod = sc_chunk[pl.ds(mg * MG + q * L, L)]
                    pid_buf[0, pl.ds(q * L, L)] = (cod & 0xFFFF) + b * T
                pltpu.async_copy(c_hbm.at[pid_buf.at[0]], crows, sem).wait()
                ng = jnp.minimum(MG // L,
                                 (j1 - (base_c + mg * MG) + L - 1) // L)

                def grp(gi, _):
                    code16 = sc_chunk[pl.ds(mg * MG + gi * L, L)]
                    s16 = (code16 >> 16) - (zi + lo)
                    base = base_c + mg * MG + gi * L
                    j0v = zi + j0
                    j1v = zi + j1
                    dumpcol = zi + MBLK
                    for jj in range(L):
                        posv = zi + (base + jj)
                        mjv = (posv >= j0v) & (posv < j1v)
                        sb = jnp.where(mjv, _vgather(s16, zi + jj), dumpcol)
                        rr = gi * L + jj
                        for g in range(CD // L):
                            fl = (g * L + iv) * (MBLK + L) + sb
                            cur = plsc.load_gather(acc, [fl])
                            val = crows[rr, pl.ds(g * L, L)]
                            plsc.store_scatter(acc, [fl], cur + val)
                        sb0 = jnp.where(lane0, sb, dumpcol)
                        ccur = plsc.load_gather(cnt_buf, [sb0])
                        plsc.store_scatter(cnt_buf, [sb0], ccur + 1.0)
                    return 0
                lax.fori_loop(0, ng, grp, 0)
                return 0
            lax.fori_loop(0, nmg, macro, 0)
            return 0
        lax.fori_loop(0, nchk, chunkf, 0)

        def fin(sg, _):
            cv = cnt_buf[pl.ds(sg * L, L)]
            inv = 1.0 / jnp.maximum(cv, 1.0)
            for ch in range(CD):
                o = ch * (MBLK + L)
                acc[pl.ds(o + sg * L, L)] = acc[pl.ds(o + sg * L, L)] * inv
            return 0
        lax.fori_loop(0, MBLK // L, fin, 0)
        for ch in range(CD):
            pltpu.sync_copy(acc.at[pl.ds(ch * (MBLK + L), MBLK)],
                            fea_hbm.at[pl.ds(b * CD * S + ch * S + lo, MBLK)])


@functools.lru_cache(maxsize=None)
def _sc_fns():
    mesh = plsc.VectorSubcoreMesh(core_axis_name="c", subcore_axis_name="s",
                                  num_cores=NC, num_subcores=NS)
    cparams = pltpu.CompilerParams(needs_layout_passes=False)
    segmax = pl.kernel(
        _sc_segmax_body, mesh=mesh, compiler_params=cparams,
        out_type=jax.ShapeDtypeStruct((B * S * HID,), jnp.float32),
        scratch_types=[
            pltpu.VMEM((L,), jnp.int32),
            pltpu.VMEM((L,), jnp.int32),
            pltpu.VMEM((CCHUNK,), jnp.int32),
            pltpu.VMEM((1, MG), jnp.int32),
            pltpu.VMEM((MG, HID), jnp.float32),
            pltpu.VMEM(((SEGBLK + 1) * HID,), jnp.float32),
            pltpu.SemaphoreType.DMA,
        ])
    gather = pl.kernel(
        _sc_gather_body, mesh=mesh, compiler_params=cparams,
        out_type=jax.ShapeDtypeStruct((BT, HID), jnp.float32),
        scratch_types=[
            pltpu.VMEM((1, _GB_SUB), jnp.int32),
            pltpu.VMEM((_GB_SUB, HID), jnp.float32),
            pltpu.SemaphoreType.DMA,
        ])
    mean = pl.kernel(
        _sc_mean_body, mesh=mesh, compiler_params=cparams,
        out_type=jax.ShapeDtypeStruct((B * CD * S,), jnp.float32),
        scratch_types=[
            pltpu.VMEM((L,), jnp.int32),
            pltpu.VMEM((L,), jnp.int32),
            pltpu.VMEM((CCHUNK,), jnp.int32),
            pltpu.VMEM((1, MG), jnp.int32),
            pltpu.VMEM((MG, HID), jnp.float32),
            pltpu.VMEM((CD * (MBLK + L),), jnp.float32),
            pltpu.VMEM((MBLK + L,), jnp.float32),
            pltpu.SemaphoreType.DMA,
        ])
    return segmax, gather, mean


def _route(index):
    idxf = index[:, 0, :].astype(jnp.int32)
    pid = lax.broadcasted_iota(jnp.int32, (B, T), 1)
    scode_bt = jnp.sort((idxf << 16) | pid, axis=1)
    scode = jnp.concatenate(
        [scode_bt.reshape(BT), jnp.zeros((CCHUNK,), jnp.int32)])

    w = np.arange(NW, dtype=np.int64)
    los = (np.arange(2, dtype=np.int64)[:, None] * NW + w[None, :]) * SEGBLK
    edges = np.minimum((np.stack([los, los + SEGBLK], axis=-1) << 16),
                       np.int64(2**31 - 1)).reshape(-1).astype(np.int32)
    edges = jnp.asarray(edges)
    jm = jax.vmap(lambda sc: jnp.searchsorted(sc, edges))(scode_bt)
    jm = (jm.astype(jnp.int32).reshape(B, 2, NW, 2)
          + (jnp.arange(B, dtype=jnp.int32) * T)[:, None, None, None])
    bm = jnp.transpose(jm, (2, 0, 1, 3)).reshape(NW, 8)
    bounds_max = jnp.concatenate(
        [bm, jnp.zeros((NW, 8), jnp.int32)], axis=1).reshape(NW * L)

    mlos = w * MBLK
    medges = np.minimum((np.stack([mlos, mlos + MBLK], axis=-1) << 16),
                        np.int64(2**31 - 1)).reshape(-1).astype(np.int32)
    medges = jnp.asarray(medges)
    jq = jax.vmap(lambda sc: jnp.searchsorted(sc, medges))(scode_bt)
    jq = (jq.astype(jnp.int32).reshape(B, NW, 2)
          + (jnp.arange(B, dtype=jnp.int32) * T)[:, None, None])
    bq = jnp.transpose(jq, (1, 0, 2)).reshape(NW, 4)
    bounds_mean = jnp.concatenate(
        [bq, jnp.zeros((NW, 12), jnp.int32)], axis=1).reshape(NW * L)

    iota = jnp.arange(L, dtype=jnp.int32)
    return idxf.reshape(BT), scode, bounds_max, bounds_mean, iota


def _conv3d(x, w, b):
    y = jax.lax.conv_general_dilated(
        x, w, (1, 1, 1), 'SAME', dimension_numbers=('NCDHW', 'OIDHW', 'NCDHW'))
    return y + b[None, :, None, None, None]


def _maxpool(x):
    return jax.lax.reduce_window(x, -jnp.inf, jax.lax.max, (1, 1, 2, 2, 2),
                                 (1, 1, 2, 2, 2), 'VALID')


def _upsample(x):
    x = jnp.repeat(x, 2, axis=2)
    x = jnp.repeat(x, 2, axis=3)
    x = jnp.repeat(x, 2, axis=4)
    return x


def kernel(points, index, fc_pos_w, fc_pos_b, blk_fc0_w, blk_fc0_b,
           blk_fc1_w, blk_fc1_b, blk_sc_w, fc_c_w, fc_c_b,
           u_e1a_w, u_e1a_b, u_e1b_w, u_e1b_b, u_e2a_w, u_e2a_b,
           u_e2b_w, u_e2b_b, u_d1a_w, u_d1a_b, u_d1b_w, u_d1b_b,
           u_out_w, u_out_b):
    idx, scode, bounds_max, bounds_mean, iota = _route(index)
    _sc_segmax, _sc_gather, _sc_mean = _sc_fns()
    pts = points.reshape(BT, DIM)
    bp = fc_pos_b.reshape(1, 2 * HID)

    net = _run_head(pts, fc_pos_w, bp, blk_fc0_w[0],
                    blk_fc0_b[0].reshape(1, HID), blk_fc1_w[0],
                    blk_fc1_b[0].reshape(1, HID), blk_sc_w[0])

    for i in range(1, NB - 1):
        seg = _sc_segmax(net, scode, bounds_max, iota).reshape(B, S, HID)
        pooled = _sc_gather(seg, idx)
        net = _run_block(net, pooled, blk_fc0_w[i],
                         blk_fc0_b[i].reshape(1, HID), blk_fc1_w[i],
                         blk_fc1_b[i].reshape(1, HID), blk_sc_w[i])

    seg = _sc_segmax(net, scode, bounds_max, iota).reshape(B, S, HID)
    pooled = _sc_gather(seg, idx)
    wc_p = jnp.pad(fc_c_w, ((0, 0), (0, HID - CD)))
    bc_p = jnp.pad(fc_c_b, (0, HID - CD)).reshape(1, HID)
    c = _run_block_last(net, pooled, blk_fc0_w[NB - 1],
                        blk_fc0_b[NB - 1].reshape(1, HID), blk_fc1_w[NB - 1],
                        blk_fc1_b[NB - 1].reshape(1, HID), blk_sc_w[NB - 1],
                        wc_p, bc_p)

    fea = _sc_mean(c, scode, bounds_mean, iota).reshape(
        B, CD, RESO, RESO, RESO)
    e1 = jax.nn.relu(_conv3d(jax.nn.relu(_conv3d(fea, u_e1a_w, u_e1a_b)),
                             u_e1b_w, u_e1b_b))
    p = _maxpool(e1)
    e2 = jax.nn.relu(_conv3d(jax.nn.relu(_conv3d(p, u_e2a_w, u_e2a_b)),
                             u_e2b_w, u_e2b_b))
    u = _upsample(e2)
    d = jnp.concatenate([u, e1], axis=1)
    d = jax.nn.relu(_conv3d(jax.nn.relu(_conv3d(d, u_d1a_w, u_d1a_b)),
                            u_d1b_w, u_d1b_b))
    out = _conv3d(d, u_out_w, u_out_b)
    return out


# speedup vs baseline: 1.1541x; 1.0005x over previous
import functools

import numpy as np

import jax
import jax.numpy as jnp
from jax import lax
from jax.experimental import pallas as pl
from jax.experimental.pallas import tpu as pltpu
from jax.experimental.pallas import tpu_sc as plsc

B, T, DIM = 2, 50000, 3
HID, CD = 128, 32
RESO = 32
S = RESO ** 3
NB = 5
BT = B * T
ROWS = 2000
GRID = BT // ROWS


def _relu(x):
    return jnp.maximum(x, 0.0)


def _dot(a, b):
    return jnp.dot(a, b, preferred_element_type=jnp.float32)


def _head_body(pts_ref, wp_ref, bp_ref, w0_ref, b0_ref, w1_ref, b1_ref,
               ws_ref, out_ref):
    p = pts_ref[...]
    h = _dot(p, wp_ref[...]) + bp_ref[...]
    net = _dot(_relu(h), w0_ref[...]) + b0_ref[...]
    dx = _dot(_relu(net), w1_ref[...]) + b1_ref[...]
    out_ref[...] = _dot(h, ws_ref[...]) + dx


def _block_body(net_ref, pooled_ref, w0_ref, b0_ref, w1_ref, b1_ref,
                ws_ref, out_ref):
    net = net_ref[...]
    pooled = pooled_ref[...]
    w0 = w0_ref[...]
    ws = ws_ref[...]
    h = (_dot(_relu(net), w0[:HID]) + _dot(_relu(pooled), w0[HID:])
         + b0_ref[...])
    dx = _dot(_relu(h), w1_ref[...]) + b1_ref[...]
    out_ref[...] = _dot(net, ws[:HID]) + _dot(pooled, ws[HID:]) + dx


def _block_last_body(net_ref, pooled_ref, w0_ref, b0_ref, w1_ref, b1_ref,
                     ws_ref, wc_ref, bc_ref, out_ref):
    net = net_ref[...]
    pooled = pooled_ref[...]
    w0 = w0_ref[...]
    ws = ws_ref[...]
    h = (_dot(_relu(net), w0[:HID]) + _dot(_relu(pooled), w0[HID:])
         + b0_ref[...])
    dx = _dot(_relu(h), w1_ref[...]) + b1_ref[...]
    out = _dot(net, ws[:HID]) + _dot(pooled, ws[HID:]) + dx
    out_ref[...] = _dot(out, wc_ref[...]) + bc_ref[...]


def _row_spec(cols):
    return pl.BlockSpec((ROWS, cols), lambda i: (i, 0))


def _full_spec(shape):
    nd = len(shape)
    return pl.BlockSpec(shape, lambda i: (0,) * nd)


def _run_head(pts, wp, bp, w0, b0, w1, b1, ws):
    return pl.pallas_call(
        _head_body,
        grid=(GRID,),
        in_specs=[_row_spec(DIM), _full_spec(wp.shape), _full_spec(bp.shape),
                  _full_spec(w0.shape), _full_spec(b0.shape),
                  _full_spec(w1.shape), _full_spec(b1.shape),
                  _full_spec(ws.shape)],
        out_specs=_row_spec(HID),
        out_shape=jax.ShapeDtypeStruct((BT, HID), jnp.float32),
    )(pts, wp, bp, w0, b0, w1, b1, ws)


def _run_block(net, pooled, w0, b0, w1, b1, ws):
    return pl.pallas_call(
        _block_body,
        grid=(GRID,),
        in_specs=[_row_spec(HID), _row_spec(HID),
                  _full_spec(w0.shape), _full_spec(b0.shape),
                  _full_spec(w1.shape), _full_spec(b1.shape),
                  _full_spec(ws.shape)],
        out_specs=_row_spec(HID),
        out_shape=jax.ShapeDtypeStruct((BT, HID), jnp.float32),
    )(net, pooled, w0, b0, w1, b1, ws)


def _run_block_last(net, pooled, w0, b0, w1, b1, ws, wc, bc):
    return pl.pallas_call(
        _block_last_body,
        grid=(GRID,),
        in_specs=[_row_spec(HID), _row_spec(HID),
                  _full_spec(w0.shape), _full_spec(b0.shape),
                  _full_spec(w1.shape), _full_spec(b1.shape),
                  _full_spec(ws.shape), _full_spec(wc.shape),
                  _full_spec(bc.shape)],
        out_specs=_row_spec(HID),
        out_shape=jax.ShapeDtypeStruct((BT, HID), jnp.float32),
    )(net, pooled, w0, b0, w1, b1, ws, wc, bc)


NC, NS, L = 2, 16, 16
NW = NC * NS
SEGBLK = 512
MBLK = 1024
CCHUNK = 512
MG = 128
NEG_INF = float('-inf')


_DN = lax.GatherDimensionNumbers(offset_dims=(), collapsed_slice_dims=(0,),
                                 start_index_map=(0,))


def _vgather(vec, idxv):
    return lax.gather(vec, idxv.reshape(L, 1), _DN, (1,),
                      mode=lax.GatherScatterMode.PROMISE_IN_BOUNDS)


def _sc_segmax_body(vals_hbm, scode_hbm, bounds_hbm, iota_hbm, seg_hbm,
                    bnd_buf, iv_buf, sc_chunk, pid_buf, rows_buf, acc, sem):
    wid = lax.axis_index("s") * NC + lax.axis_index("c")
    woff = pl.multiple_of(wid * L, 8)
    pltpu.sync_copy(bounds_hbm.at[pl.ds(woff, L)], bnd_buf)
    pltpu.sync_copy(iota_hbm, iv_buf)
    bv = bnd_buf[...]
    iv = iv_buf[...]
    zi = iv ^ iv
    neg_row = zi.astype(jnp.float32) + NEG_INF

    for t in range(2 * B):
        b = t // 2
        lo = ((t % 2) * NW + wid) * SEGBLK
        j0 = bv[2 * t]
        j1 = bv[2 * t + 1]

        def initrow(i, _):
            acc[pl.ds(i * L, L)] = neg_row
            return 0
        lax.fori_loop(0, (SEGBLK + 1) * HID // L, initrow, 0)

        ja = (j0 // 8) * 8
        nchk = (j1 - ja + CCHUNK - 1) // CCHUNK
        dumprow = zi + SEGBLK
        carry0 = (dumprow,) + (neg_row,) * (HID // L)

        def chunkf(ci, car):
            base_c = ja + ci * CCHUNK
            coff = pl.multiple_of(base_c, 8)
            pltpu.sync_copy(scode_hbm.at[pl.ds(coff, CCHUNK)], sc_chunk)
            nmg = jnp.minimum(CCHUNK // MG, (j1 - base_c + MG - 1) // MG)

            def macro(mg, car):
                for q in range(MG // L):
                    cod = sc_chunk[pl.ds(mg * MG + q * L, L)]
                    pid_buf[0, pl.ds(q * L, L)] = (cod & 0xFFFF) + b * T
                pltpu.async_copy(vals_hbm.at[pid_buf.at[0]], rows_buf,
                                 sem).wait()
                ng = jnp.minimum(MG // L,
                                 (j1 - (base_c + mg * MG) + L - 1) // L)

                def grp(gi, car):
                    pseg = car[0]
                    run = list(car[1:])
                    code16 = sc_chunk[pl.ds(mg * MG + gi * L, L)]
                    s16 = (code16 >> 16) - (zi + lo)
                    base = base_c + mg * MG + gi * L
                    j0v = zi + j0
                    j1v = zi + j1
                    for jj in range(L):
                        posv = zi + (base + jj)
                        mjv = (posv >= j0v) & (posv < j1v)
                        sb = jnp.where(mjv, _vgather(s16, zi + jj), dumprow)
                        sel = sb == pseg
                        sbase = sb * HID
                        rr = gi * L + jj
                        for g in range(HID // L):
                            val = rows_buf[rr, pl.ds(g * L, L)]
                            run[g] = jnp.where(
                                sel, jnp.maximum(run[g], val), val)
                            plsc.store_scatter(acc, [sbase + (g * L + iv)],
                                               run[g])
                        pseg = sb
                    return (pseg,) + tuple(run)
                return lax.fori_loop(0, ng, grp, car)
            return lax.fori_loop(0, nmg, macro, car)
        lax.fori_loop(0, nchk, chunkf, carry0)
        pltpu.sync_copy(acc.at[pl.ds(0, SEGBLK * HID)],
                        seg_hbm.at[pl.ds(b * S * HID + lo * HID,
                                         SEGBLK * HID)])


_GB_SUB = 128
_GB_PER_B = (T + _GB_SUB - 1) // _GB_SUB
_GB_TOTAL = B * _GB_PER_B
_GB_ITERS = (_GB_TOTAL + NW - 1) // NW


def _sc_gather_body(seg_hbm, idx_hbm, out_hbm, idx_row, rows_buf, sem):
    wid = lax.axis_index("s") * NC + lax.axis_index("c")

    def it(i, _):
        tid = wid + i * NW

        @pl.when(tid < _GB_TOTAL)
        def _():
            bb = tid // _GB_PER_B
            j = tid % _GB_PER_B
            start = jnp.where(j == _GB_PER_B - 1, T - _GB_SUB, j * _GB_SUB)
            off = pl.multiple_of(bb * T + start, 8)
            pltpu.sync_copy(idx_hbm.at[pl.ds(off, _GB_SUB)],
                            idx_row.at[0])
            pltpu.async_copy(seg_hbm.at[bb].at[idx_row.at[0]],
                             rows_buf, sem).wait()
            pltpu.sync_copy(rows_buf,
                            out_hbm.at[pl.ds(bb * T + start, _GB_SUB), :])
        return 0
    lax.fori_loop(0, _GB_ITERS, it, 0)


def _sc_mean_body(c_hbm, scode_hbm, bounds_hbm, iota_hbm, fea_hbm, bnd_buf,
                  iv_buf, sc_chunk, pid_buf, crows, acc, cnt_buf, sem):
    wid = lax.axis_index("s") * NC + lax.axis_index("c")
    woff = pl.multiple_of(wid * L, 8)
    pltpu.sync_copy(bounds_hbm.at[pl.ds(woff, L)], bnd_buf)
    pltpu.sync_copy(iota_hbm, iv_buf)
    bv = bnd_buf[...]
    iv = iv_buf[...]
    zi = iv ^ iv
    zrow = zi.astype(jnp.float32)
    lane0 = iv < 1

    for b in range(B):
        lo = wid * MBLK
        j0 = bv[2 * b]
        j1 = bv[2 * b + 1]

        def initrow(i, _):
            acc[pl.ds(i * L, L)] = zrow
            return 0
        lax.fori_loop(0, CD * (MBLK + L) // L, initrow, 0)

        def initcnt(i, _):
            cnt_buf[pl.ds(i * L, L)] = zrow
            return 0
        lax.fori_loop(0, (MBLK + L) // L, initcnt, 0)

        ja = (j0 // 8) * 8
        nchk = (j1 - ja + CCHUNK - 1) // CCHUNK

        def chunkf(ci, _):
            base_c = ja + ci * CCHUNK
            coff = pl.multiple_of(base_c, 8)
            pltpu.sync_copy(scode_hbm.at[pl.ds(coff, CCHUNK)], sc_chunk)
            nmg = jnp.minimum(CCHUNK // MG, (j1 - base_c + MG - 1) // MG)

            def macro(mg, _):
                for q in range(MG // L):
                    cod = sc_chunk[pl.ds(mg * MG + q * L, L)]
                    pid_buf[0, pl.ds(q * L, L)] = (cod & 0xFFFF) + b * T
                pltpu.async_copy(c_hbm.at[pid_buf.at[0]], crows, sem).wait()
                ng = jnp.minimum(MG // L,
                                 (j1 - (base_c + mg * MG) + L - 1) // L)

                def grp(gi, _):
                    code16 = sc_chunk[pl.ds(mg * MG + gi * L, L)]
                    s16 = (code16 >> 16) - (zi + lo)
                    base = base_c + mg * MG + gi * L
                    j0v = zi + j0
                    j1v = zi + j1
                    dumpcol = zi + MBLK
                    for jj in range(L):
                        posv = zi + (base + jj)
                        mjv = (posv >= j0v) & (posv < j1v)
                        sb = jnp.where(mjv, _vgather(s16, zi + jj), dumpcol)
                        rr = gi * L + jj
                        for g in range(CD // L):
                            fl = (g * L + iv) * (MBLK + L) + sb
                            cur = plsc.load_gather(acc, [fl])
                            val = crows[rr, pl.ds(g * L, L)]
                            plsc.store_scatter(acc, [fl], cur + val)
                        sb0 = jnp.where(lane0, sb, dumpcol)
                        ccur = plsc.load_gather(cnt_buf, [sb0])
                        plsc.store_scatter(cnt_buf, [sb0], ccur + 1.0)
                    return 0
                lax.fori_loop(0, ng, grp, 0)
                return 0
            lax.fori_loop(0, nmg, macro, 0)
            return 0
        lax.fori_loop(0, nchk, chunkf, 0)

        def fin(sg, _):
            cv = cnt_buf[pl.ds(sg * L, L)]
            inv = 1.0 / jnp.maximum(cv, 1.0)
            for ch in range(CD):
                o = ch * (MBLK + L)
                acc[pl.ds(o + sg * L, L)] = acc[pl.ds(o + sg * L, L)] * inv
            return 0
        lax.fori_loop(0, MBLK // L, fin, 0)
        for ch in range(CD):
            pltpu.sync_copy(acc.at[pl.ds(ch * (MBLK + L), MBLK)],
                            fea_hbm.at[pl.ds(b * CD * S + ch * S + lo, MBLK)])


@functools.lru_cache(maxsize=None)
def _sc_fns():
    mesh = plsc.VectorSubcoreMesh(core_axis_name="c", subcore_axis_name="s",
                                  num_cores=NC, num_subcores=NS)
    cparams = pltpu.CompilerParams(needs_layout_passes=False)
    segmax = pl.kernel(
        _sc_segmax_body, mesh=mesh, compiler_params=cparams,
        out_type=jax.ShapeDtypeStruct((B * S * HID,), jnp.float32),
        scratch_types=[
            pltpu.VMEM((L,), jnp.int32),
            pltpu.VMEM((L,), jnp.int32),
            pltpu.VMEM((CCHUNK,), jnp.int32),
            pltpu.VMEM((1, MG), jnp.int32),
            pltpu.VMEM((MG, HID), jnp.float32),
            pltpu.VMEM(((SEGBLK + 1) * HID,), jnp.float32),
            pltpu.SemaphoreType.DMA,
        ])
    gather = pl.kernel(
        _sc_gather_body, mesh=mesh, compiler_params=cparams,
        out_type=jax.ShapeDtypeStruct((BT, HID), jnp.float32),
        scratch_types=[
            pltpu.VMEM((1, _GB_SUB), jnp.int32),
            pltpu.VMEM((_GB_SUB, HID), jnp.float32),
            pltpu.SemaphoreType.DMA,
        ])
    mean = pl.kernel(
        _sc_mean_body, mesh=mesh, compiler_params=cparams,
        out_type=jax.ShapeDtypeStruct((B * CD * S,), jnp.float32),
        scratch_types=[
            pltpu.VMEM((L,), jnp.int32),
            pltpu.VMEM((L,), jnp.int32),
            pltpu.VMEM((CCHUNK,), jnp.int32),
            pltpu.VMEM((1, MG), jnp.int32),
            pltpu.VMEM((MG, HID), jnp.float32),
            pltpu.VMEM((CD * (MBLK + L),), jnp.float32),
            pltpu.VMEM((MBLK + L,), jnp.float32),
            pltpu.SemaphoreType.DMA,
        ])
    return segmax, gather, mean


def _route(index):
    idxf = index[:, 0, :].astype(jnp.int32)
    pid = lax.broadcasted_iota(jnp.int32, (B, T), 1)
    scode_bt = jnp.sort((idxf << 16) | pid, axis=1)
    scode = jnp.concatenate(
        [scode_bt.reshape(BT), jnp.zeros((CCHUNK,), jnp.int32)])

    w = np.arange(NW, dtype=np.int64)
    los = (np.arange(2, dtype=np.int64)[:, None] * NW + w[None, :]) * SEGBLK
    edges = np.minimum((np.stack([los, los + SEGBLK], axis=-1) << 16),
                       np.int64(2**31 - 1)).reshape(-1).astype(np.int32)
    edges = jnp.asarray(edges)
    jm = jax.vmap(lambda sc: jnp.searchsorted(sc, edges))(scode_bt)
    jm = (jm.astype(jnp.int32).reshape(B, 2, NW, 2)
          + (jnp.arange(B, dtype=jnp.int32) * T)[:, None, None, None])
    bm = jnp.transpose(jm, (2, 0, 1, 3)).reshape(NW, 8)
    bounds_max = jnp.concatenate(
        [bm, jnp.zeros((NW, 8), jnp.int32)], axis=1).reshape(NW * L)

    mlos = w * MBLK
    medges = np.minimum((np.stack([mlos, mlos + MBLK], axis=-1) << 16),
                        np.int64(2**31 - 1)).reshape(-1).astype(np.int32)
    medges = jnp.asarray(medges)
    jq = jax.vmap(lambda sc: jnp.searchsorted(sc, medges))(scode_bt)
    jq = (jq.astype(jnp.int32).reshape(B, NW, 2)
          + (jnp.arange(B, dtype=jnp.int32) * T)[:, None, None])
    bq = jnp.transpose(jq, (1, 0, 2)).reshape(NW, 4)
    bounds_mean = jnp.concatenate(
        [bq, jnp.zeros((NW, 12), jnp.int32)], axis=1).reshape(NW * L)

    iota = jnp.arange(L, dtype=jnp.int32)
    return idxf.reshape(BT), scode, bounds_max, bounds_mean, iota


def _conv3d(x, w, b):
    y = jax.lax.conv_general_dilated(
        x, w, (1, 1, 1), 'SAME', dimension_numbers=('NCDHW', 'OIDHW', 'NCDHW'))
    return y + b[None, :, None, None, None]


def _maxpool(x):
    return jax.lax.reduce_window(x, -jnp.inf, jax.lax.max, (1, 1, 2, 2, 2),
                                 (1, 1, 2, 2, 2), 'VALID')


def _upsample(x):
    x = jnp.repeat(x, 2, axis=2)
    x = jnp.repeat(x, 2, axis=3)
    x = jnp.repeat(x, 2, axis=4)
    return x


def kernel(points, index, fc_pos_w, fc_pos_b, blk_fc0_w, blk_fc0_b,
           blk_fc1_w, blk_fc1_b, blk_sc_w, fc_c_w, fc_c_b,
           u_e1a_w, u_e1a_b, u_e1b_w, u_e1b_b, u_e2a_w, u_e2a_b,
           u_e2b_w, u_e2b_b, u_d1a_w, u_d1a_b, u_d1b_w, u_d1b_b,
           u_out_w, u_out_b):
    idx, scode, bounds_max, bounds_mean, iota = _route(index)
    _sc_segmax, _sc_gather, _sc_mean = _sc_fns()
    pts = points.reshape(BT, DIM)
    bp = fc_pos_b.reshape(1, 2 * HID)

    net = _run_head(pts, fc_pos_w, bp, blk_fc0_w[0],
                    blk_fc0_b[0].reshape(1, HID), blk_fc1_w[0],
                    blk_fc1_b[0].reshape(1, HID), blk_sc_w[0])

    for i in range(1, NB - 1):
        seg = _sc_segmax(net, scode, bounds_max, iota).reshape(B, S, HID)
        pooled = _sc_gather(seg, idx)
        net = _run_block(net, pooled, blk_fc0_w[i],
                         blk_fc0_b[i].reshape(1, HID), blk_fc1_w[i],
                         blk_fc1_b[i].reshape(1, HID), blk_sc_w[i])

    seg = _sc_segmax(net, scode, bounds_max, iota).reshape(B, S, HID)
    pooled = _sc_gather(seg, idx)
    wc_p = jnp.pad(fc_c_w, ((0, 0), (0, HID - CD)))
    bc_p = jnp.pad(fc_c_b, (0, HID - CD)).reshape(1, HID)
    c = _run_block_last(net, pooled, blk_fc0_w[NB - 1],
                        blk_fc0_b[NB - 1].reshape(1, HID), blk_fc1_w[NB - 1],
                        blk_fc1_b[NB - 1].reshape(1, HID), blk_sc_w[NB - 1],
                        wc_p, bc_p)

    fea = _sc_mean(c, scode, bounds_mean, iota).reshape(
        B, CD, RESO, RESO, RESO)
    e1 = jax.nn.relu(_conv3d(jax.nn.relu(_conv3d(fea, u_e1a_w, u_e1a_b)),
                             u_e1b_w, u_e1b_b))
    p = _maxpool(e1)
    e2 = jax.nn.relu(_conv3d(jax.nn.relu(_conv3d(p, u_e2a_w, u_e2a_b)),
                             u_e2b_w, u_e2b_b))
    u = _upsample(e2)
    d = jnp.concatenate([u, e1], axis=1)
    d = jax.nn.relu(_conv3d(jax.nn.relu(_conv3d(d, u_d1a_w, u_d1a_b)),
                            u_d1b_w, u_d1b_b))
    out = _conv3d(d, u_out_w, u_out_b)
    return out
